# baseline (speedup 1.0000x reference)
import jax
import jax.numpy as jnp
from jax.experimental import pallas as pl
from jax.experimental.pallas import tpu as pltpu
from jax.experimental.pallas import tpu_sc as plsc

_PB = 16


def _fma_body(t_ref, a_ref, c_ref, x_ref, eps_ref, x0_ref):
    g = pl.program_id(0)
    for j in range(_PB):
        ti = t_ref[g * _PB + j]
        x0_ref[j] = a_ref[ti] * x_ref[j] - c_ref[ti] * eps_ref[j]


def _tc_fma(x_t, model_preds, t, srac, srm1ac):
    B, C, H, W = x_t.shape
    blk = pl.BlockSpec((_PB, C, H, W), lambda g, *_: (g, 0, 0, 0))
    grid_spec = pltpu.PrefetchScalarGridSpec(
        num_scalar_prefetch=3,
        grid=(B // _PB,),
        in_specs=[blk, blk],
        out_specs=blk,
    )
    return pl.pallas_call(
        _fma_body,
        grid_spec=grid_spec,
        out_shape=jax.ShapeDtypeStruct(x_t.shape, x_t.dtype),
    )(t, srac, srm1ac, x_t, model_preds)


def _sc_copies(model_preds, noise):
    mesh = plsc.ScalarSubcoreMesh(axis_name="core", num_cores=2)
    out = jax.ShapeDtypeStruct(model_preds.shape, model_preds.dtype)

    @pl.kernel(out_type=(out, out), mesh=mesh,
               scratch_types=[pltpu.SemaphoreType.DMA,
                              pltpu.SemaphoreType.DMA])
    def sc_copy(eps_hbm, nz_hbm, np_hbm, tg_hbm, sem1, sem2):
        idx = jax.lax.axis_index("core")

        @pl.when(idx == 0)
        def _():
            pltpu.async_copy(eps_hbm, np_hbm, sem1).wait()

        @pl.when(idx == 1)
        def _():
            pltpu.async_copy(nz_hbm, tg_hbm, sem2).wait()

    return sc_copy(model_preds, noise)


def kernel(model_preds, x_t, x_0, noise, t,
           sqrt_recip_alphas_cumprod, sqrt_recipm1_alphas_cumprod):
    x0p = _tc_fma(x_t, model_preds, t,
                  sqrt_recip_alphas_cumprod, sqrt_recipm1_alphas_cumprod)
    np_, tg = _sc_copies(model_preds, noise)
    return (np_, x0p, tg)


# speedup vs baseline: 1.0490x; 1.0490x over previous
import jax
import jax.numpy as jnp
from jax.experimental import pallas as pl
from jax.experimental.pallas import tpu as pltpu

_PB = 16


def _body(t_ref, a_ref, c_ref, x_ref, eps_ref, eps_hbm, nz_hbm,
          x0_ref, np_hbm, tg_hbm, sem1, sem2):
    g = pl.program_id(0)

    @pl.when(g == 0)
    def _():
        pltpu.make_async_copy(eps_hbm, np_hbm, sem1).start()
        pltpu.make_async_copy(nz_hbm, tg_hbm, sem2).start()

    for j in range(_PB):
        ti = t_ref[g * _PB + j]
        x0_ref[j] = a_ref[ti] * x_ref[j] - c_ref[ti] * eps_ref[j]

    @pl.when(g == pl.num_programs(0) - 1)
    def _():
        pltpu.make_async_copy(eps_hbm, np_hbm, sem1).wait()
        pltpu.make_async_copy(nz_hbm, tg_hbm, sem2).wait()


def kernel(model_preds, x_t, x_0, noise, t,
           sqrt_recip_alphas_cumprod, sqrt_recipm1_alphas_cumprod):
    B, C, H, W = x_t.shape
    blk = pl.BlockSpec((_PB, C, H, W), lambda g, *_: (g, 0, 0, 0))
    anyspec = pl.BlockSpec(memory_space=pl.ANY)
    grid_spec = pltpu.PrefetchScalarGridSpec(
        num_scalar_prefetch=3,
        grid=(B // _PB,),
        in_specs=[blk, blk, anyspec, anyspec],
        out_specs=[blk, anyspec, anyspec],
        scratch_shapes=[pltpu.SemaphoreType.DMA, pltpu.SemaphoreType.DMA],
    )
    out = jax.ShapeDtypeStruct(x_t.shape, x_t.dtype)
    x0p, np_, tg = pl.pallas_call(
        _body,
        grid_spec=grid_spec,
        out_shape=[out, out, out],
    )(t, sqrt_recip_alphas_cumprod, sqrt_recipm1_alphas_cumprod,
      x_t, model_preds, model_preds, noise)
    return (np_, x0p, tg)


# speedup vs baseline: 10.3500x; 9.8664x over previous
import jax
import jax.numpy as jnp
from jax.experimental import pallas as pl
from jax.experimental.pallas import tpu as pltpu
from jax.experimental.pallas import tpu_sc as plsc

_PB = 16


def _body(t_ref, a_ref, c_ref, x_ref, eps_ref, x0_ref, np_ref):
    g = pl.program_id(0)
    np_ref[...] = eps_ref[...]
    for j in range(_PB):
        ti = t_ref[g * _PB + j]
        x0_ref[j] = a_ref[ti] * x_ref[j] - c_ref[ti] * eps_ref[j]


def _tc_part(x_t, model_preds, t, srac, srm1ac):
    B, C, H, W = x_t.shape
    blk = pl.BlockSpec((_PB, C, H, W), lambda g, *_: (g, 0, 0, 0))
    grid_spec = pltpu.PrefetchScalarGridSpec(
        num_scalar_prefetch=3,
        grid=(B // _PB,),
        in_specs=[blk, blk],
        out_specs=[blk, blk],
    )
    out = jax.ShapeDtypeStruct(x_t.shape, x_t.dtype)
    return pl.pallas_call(
        _body,
        grid_spec=grid_spec,
        out_shape=[out, out],
    )(t, srac, srm1ac, x_t, model_preds)


def _sc_copy(noise):
    mesh = plsc.VectorSubcoreMesh(core_axis_name="core",
                                  subcore_axis_name="subcore")
    B, C, H, W = noise.shape

    @pl.kernel(out_type=jax.ShapeDtypeStruct(noise.shape, noise.dtype),
               mesh=mesh,
               scratch_types=[pltpu.VMEM((1, C, H, W), jnp.float32),
                              pltpu.SemaphoreType.DMA,
                              pltpu.SemaphoreType.DMA])
    def sc_copy(nz_hbm, tg_hbm, buf, insem, outsem):
        core = jax.lax.axis_index("core")
        sub = jax.lax.axis_index("subcore")
        wid = core * mesh.num_subcores + sub
        pltpu.async_copy(nz_hbm.at[pl.ds(wid, 1)], buf, insem).wait()
        pltpu.async_copy(buf, tg_hbm.at[pl.ds(wid, 1)], outsem).wait()

    return sc_copy(noise)


def kernel(model_preds, x_t, x_0, noise, t,
           sqrt_recip_alphas_cumprod, sqrt_recipm1_alphas_cumprod):
    x0p, np_ = _tc_part(x_t, model_preds, t,
                        sqrt_recip_alphas_cumprod, sqrt_recipm1_alphas_cumprod)
    tg = _sc_copy(noise)
    return (np_, x0p, tg)


# speedup vs baseline: 27.6338x; 2.6699x over previous
import jax
import jax.numpy as jnp
from jax.experimental import pallas as pl
from jax.experimental.pallas import tpu as pltpu

_PB = 16


def _body(t_ref, a_ref, c_ref, x_ref, eps_ref, nz_ref, x0_ref, np_ref, tg_ref):
    g = pl.program_id(0)
    np_ref[...] = eps_ref[...]
    tg_ref[...] = nz_ref[...]
    for j in range(_PB):
        ti = t_ref[g * _PB + j]
        x0_ref[j] = a_ref[ti] * x_ref[j] - c_ref[ti] * eps_ref[j]


def kernel(model_preds, x_t, x_0, noise, t,
           sqrt_recip_alphas_cumprod, sqrt_recipm1_alphas_cumprod):
    B, C, H, W = x_t.shape
    blk = pl.BlockSpec((_PB, C, H, W), lambda g, *_: (g, 0, 0, 0))
    grid_spec = pltpu.PrefetchScalarGridSpec(
        num_scalar_prefetch=3,
        grid=(B // _PB,),
        in_specs=[blk, blk, blk],
        out_specs=[blk, blk, blk],
    )
    out = jax.ShapeDtypeStruct(x_t.shape, x_t.dtype)
    x0p, np_, tg = pl.pallas_call(
        _body,
        grid_spec=grid_spec,
        out_shape=[out, out, out],
    )(t, sqrt_recip_alphas_cumprod, sqrt_recipm1_alphas_cumprod,
      x_t, model_preds, noise)
    return (np_, x0p, tg)
